# baseline (speedup 1.0000x reference)
import jax
import jax.numpy as jnp
from jax import lax
from jax.experimental import pallas as pl
from jax.experimental.pallas import tpu as pltpu
from jax.experimental.pallas import tpu_sc as plsc

B = 16384
L = 200
D = 32
NC = 2
NS = 16
NW = NC * NS
UW = B // NW
CU = 8
NCHUNK = UW // CU
LH = 100


def _tec_body(users_hbm, pred_hbm, iemb_hbm, uemb_hbm, out_hbm,
              uidx, usup, idx, urows, irows, outv, sem_s, sem_g, sem_w):
    wid = lax.axis_index("s") * NC + lax.axis_index("c")
    base = wid * UW
    lanes = lax.iota(jnp.int32, 16)
    ulane = lanes & (CU - 1)
    srow = lanes >> 3
    lo8 = lanes < 8

    def stage_start(c, s):
        row0 = base + c * CU
        pltpu.async_copy(users_hbm.at[pl.ds(row0, CU)], uidx[s], sem_s[s])
        pltpu.async_copy(pred_hbm.at[pl.ds(row0, CU)], idx[s], sem_s[s])

    def stage_wait(s):
        pltpu.make_async_copy(users_hbm.at[pl.ds(0, CU)], uidx[s], sem_s[s]).wait()
        pltpu.make_async_copy(pred_hbm.at[pl.ds(0, CU)], idx[s], sem_s[s]).wait()

    def gathers_start(s):
        ul = plsc.load_gather(uidx[s], [ulane])
        plsc.store_scatter(usup[s], [ulane], ul >> 2, mask=lo8)
        pltpu.async_copy(uemb_hbm.at[usup[s]], urows[s], sem_g[s])
        for j in range(CU):
            for h in range(2):
                pltpu.async_copy(
                    iemb_hbm.at[idx[s].at[j, h]],
                    irows[s].at[pl.ds((j * 2 + h) * LH, LH)],
                    sem_g[s])

    def gathers_wait(s):
        pltpu.make_async_copy(uemb_hbm.at[pl.ds(0, CU)], urows[s], sem_g[s]).wait()
        pltpu.make_async_copy(iemb_hbm.at[pl.ds(0, CU * L)], irows[s], sem_g[s]).wait()

    def writeback_start(c, s):
        row0 = base + c * CU
        pltpu.async_copy(outv[s], out_hbm.at[pl.ds(row0, CU)], sem_w[s])

    def writeback_wait(s):
        pltpu.make_async_copy(outv[s], out_hbm.at[pl.ds(0, CU)], sem_w[s]).wait()

    def compute(s):
        ir = irows[s]
        ur = urows[s]
        ov = outv[s]
        uload = plsc.load_gather(uidx[s], [ulane])
        uoff = (uload & 3) * D

        @plsc.parallel_loop(0, L, step=2, unroll=16)
        def slot_body(l):
            rows = ulane * L + (l + srow)
            acc = jnp.zeros((16,), jnp.float32)
            for d in range(D):
                col = (lanes + d) & (D - 1)
                iv = plsc.load_gather(ir, [rows, col])
                uv = plsc.load_gather(ur, [ulane, uoff + col])
                t = uv - iv
                acc = acc + t * t
            plsc.store_scatter(ov, [ulane, l + srow], -acc)

    stage_start(0, 0)
    stage_wait(0)
    gathers_start(0)
    stage_start(1, 1)

    def pair_body(k, _):
        for s in (0, 1):
            c = 2 * k + s
            gathers_wait(s)

            @pl.when(c + 1 < NCHUNK)
            def _():
                stage_wait(s ^ 1)
                gathers_start(s ^ 1)

            @pl.when(c + 2 < NCHUNK)
            def _():
                stage_start(c + 2, s)

            @pl.when(c >= 2)
            def _():
                writeback_wait(s)

            compute(s)
            writeback_start(c, s)
        return _

    lax.fori_loop(0, NCHUNK // 2, pair_body, None)
    writeback_wait(0)
    writeback_wait(1)


def kernel(users, pred_items, i_emb_weight, u_emb_weight):
    mesh = plsc.VectorSubcoreMesh(core_axis_name="c", subcore_axis_name="s")

    def body(users_hbm, pred_hbm, iemb_hbm, uemb_hbm, out_hbm,
             uidx0, uidx1, usup0, usup1, idx0, idx1, urows0, urows1,
             irows0, irows1,
             out0, out1, sem_s0, sem_s1, sem_g0, sem_g1, sem_w0, sem_w1):
        _tec_body(users_hbm, pred_hbm, iemb_hbm, uemb_hbm, out_hbm,
                  (uidx0, uidx1), (usup0, usup1), (idx0, idx1),
                  (urows0, urows1),
                  (irows0, irows1), (out0, out1),
                  (sem_s0, sem_s1), (sem_g0, sem_g1), (sem_w0, sem_w1))

    k = pl.kernel(
        body,
        out_type=jax.ShapeDtypeStruct((B, L), jnp.float32),
        mesh=mesh,
        compiler_params=pltpu.CompilerParams(
            needs_layout_passes=False, use_tc_tiling_on_sc=False),
        scratch_types=[
            pltpu.VMEM((CU,), jnp.int32),
            pltpu.VMEM((CU,), jnp.int32),
            pltpu.VMEM((CU,), jnp.int32),
            pltpu.VMEM((CU,), jnp.int32),
            pltpu.VMEM((CU, 2, LH), jnp.int32),
            pltpu.VMEM((CU, 2, LH), jnp.int32),
            pltpu.VMEM((CU, 128), jnp.float32),
            pltpu.VMEM((CU, 128), jnp.float32),
            pltpu.VMEM((CU * L, D), jnp.float32),
            pltpu.VMEM((CU * L, D), jnp.float32),
            pltpu.VMEM((CU, L), jnp.float32),
            pltpu.VMEM((CU, L), jnp.float32),
            pltpu.SemaphoreType.DMA,
            pltpu.SemaphoreType.DMA,
            pltpu.SemaphoreType.DMA,
            pltpu.SemaphoreType.DMA,
            pltpu.SemaphoreType.DMA,
            pltpu.SemaphoreType.DMA,
        ],
    )
    return k(users, pred_items.reshape(B, 2, LH), i_emb_weight,
             u_emb_weight.reshape(250000, 128))


# speedup vs baseline: 1.6102x; 1.6102x over previous
import jax
import jax.numpy as jnp
from jax import lax
from jax.experimental import pallas as pl
from jax.experimental.pallas import tpu as pltpu
from jax.experimental.pallas import tpu_sc as plsc

B = 16384
L = 200
D = 32
NC = 2
NS = 16
NW = NC * NS
UW = B // NW
CU = 8
NCHUNK = UW // CU
LH = 100


def _tec_body(users_hbm, pred_hbm, iemb_hbm, uemb_hbm, out_hbm,
              uidx, usup, idx, urows, irows, outv, sem_s, sem_g, sem_w):
    wid = lax.axis_index("s") * NC + lax.axis_index("c")
    base = wid * UW
    lanes = lax.iota(jnp.int32, 16)
    ulane = lanes & (CU - 1)
    srow = lanes >> 3
    lo8 = lanes < 8

    def stage_start(c, s):
        row0 = base + c * CU
        pltpu.async_copy(users_hbm.at[pl.ds(row0, CU)], uidx[s], sem_s[s])
        pltpu.async_copy(pred_hbm.at[pl.ds(row0, CU)], idx[s], sem_s[s])

    def stage_wait(s):
        pltpu.make_async_copy(users_hbm.at[pl.ds(0, CU)], uidx[s], sem_s[s]).wait()
        pltpu.make_async_copy(pred_hbm.at[pl.ds(0, CU)], idx[s], sem_s[s]).wait()

    def gathers_start(s):
        ul = plsc.load_gather(uidx[s], [ulane])
        plsc.store_scatter(usup[s], [ulane], ul >> 2, mask=lo8)
        pltpu.async_copy(uemb_hbm.at[usup[s]], urows[s], sem_g[s])
        for j in range(CU):
            for h in range(2):
                pltpu.async_copy(
                    iemb_hbm.at[idx[s].at[j, h]],
                    irows[s].at[pl.ds((j * 2 + h) * LH, LH)],
                    sem_g[s])

    def gathers_wait(s):
        pltpu.make_async_copy(uemb_hbm.at[pl.ds(0, CU)], urows[s], sem_g[s]).wait()
        pltpu.make_async_copy(iemb_hbm.at[pl.ds(0, CU * L)], irows[s], sem_g[s]).wait()

    def writeback_start(c, s):
        row0 = base + c * CU
        pltpu.async_copy(outv[s], out_hbm.at[pl.ds(row0, CU)], sem_w[s])

    def writeback_wait(s):
        pltpu.make_async_copy(outv[s], out_hbm.at[pl.ds(0, CU)], sem_w[s]).wait()

    def compute(s):
        ir = irows[s]
        ur = urows[s]
        ov = outv[s]
        uload = plsc.load_gather(uidx[s], [ulane])
        uoff = (uload & 3) * D

        @plsc.parallel_loop(0, L, step=2, unroll=8)
        def slot_body(l):
            rows = ulane * L + (l + srow)
            acc = jnp.zeros((16,), jnp.float32)
            for d in range(D):
                col = (lanes + d) & (D - 1)
                iv = plsc.load_gather(ir, [rows, col])
                uv = plsc.load_gather(ur, [ulane, uoff + col])
                t = uv - iv
                acc = acc + t * t
            plsc.store_scatter(ov, [ulane, l + srow], -acc)

    stage_start(0, 0)
    stage_wait(0)
    gathers_start(0)
    stage_start(1, 1)

    def pair_body(k, _):
        for s in (0, 1):
            c = 2 * k + s
            gathers_wait(s)

            @pl.when(c + 1 < NCHUNK)
            def _():
                stage_wait(s ^ 1)
                gathers_start(s ^ 1)

            @pl.when(c + 2 < NCHUNK)
            def _():
                stage_start(c + 2, s)

            @pl.when(c >= 2)
            def _():
                writeback_wait(s)

            compute(s)
            writeback_start(c, s)
        return _

    lax.fori_loop(0, NCHUNK // 2, pair_body, None)
    writeback_wait(0)
    writeback_wait(1)


def kernel(users, pred_items, i_emb_weight, u_emb_weight):
    mesh = plsc.VectorSubcoreMesh(core_axis_name="c", subcore_axis_name="s")

    def body(users_hbm, pred_hbm, iemb_hbm, uemb_hbm, out_hbm,
             uidx0, uidx1, usup0, usup1, idx0, idx1, urows0, urows1,
             irows0, irows1,
             out0, out1, sem_s0, sem_s1, sem_g0, sem_g1, sem_w0, sem_w1):
        _tec_body(users_hbm, pred_hbm, iemb_hbm, uemb_hbm, out_hbm,
                  (uidx0, uidx1), (usup0, usup1), (idx0, idx1),
                  (urows0, urows1),
                  (irows0, irows1), (out0, out1),
                  (sem_s0, sem_s1), (sem_g0, sem_g1), (sem_w0, sem_w1))

    k = pl.kernel(
        body,
        out_type=jax.ShapeDtypeStruct((B, L), jnp.float32),
        mesh=mesh,
        compiler_params=pltpu.CompilerParams(
            needs_layout_passes=False, use_tc_tiling_on_sc=False),
        scratch_types=[
            pltpu.VMEM((CU,), jnp.int32),
            pltpu.VMEM((CU,), jnp.int32),
            pltpu.VMEM((CU,), jnp.int32),
            pltpu.VMEM((CU,), jnp.int32),
            pltpu.VMEM((CU, 2, LH), jnp.int32),
            pltpu.VMEM((CU, 2, LH), jnp.int32),
            pltpu.VMEM((CU, 128), jnp.float32),
            pltpu.VMEM((CU, 128), jnp.float32),
            pltpu.VMEM((CU * L, D), jnp.float32),
            pltpu.VMEM((CU * L, D), jnp.float32),
            pltpu.VMEM((CU, L), jnp.float32),
            pltpu.VMEM((CU, L), jnp.float32),
            pltpu.SemaphoreType.DMA,
            pltpu.SemaphoreType.DMA,
            pltpu.SemaphoreType.DMA,
            pltpu.SemaphoreType.DMA,
            pltpu.SemaphoreType.DMA,
            pltpu.SemaphoreType.DMA,
        ],
    )
    return k(users, pred_items.reshape(B, 2, LH), i_emb_weight,
             u_emb_weight.reshape(250000, 128))
